# Initial kernel scaffold; baseline (speedup 1.0000x reference)
import jax
import jax.numpy as jnp
from jax.experimental import pallas as pl


def kernel(x, edge_index, y, pmp_mask, W_fr1, W_be1, Wa1, ba1, W_self1, b_self1, W_fr2, W_be2, Wa2, ba2, W_self2, b_self2, Wc, bc):
    raise NotImplementedError("write your pallas kernel here")


# speedup vs baseline: 12.2954x; 12.2954x over previous
import functools

import jax
import jax.numpy as jnp
from jax import lax
from jax.experimental import pallas as pl
from jax.experimental.pallas import tpu as pltpu
from jax.experimental.pallas import tpu_sc as plsc

N = 10000
E = 320000
D = 128
ODIM = 16

NC = 2
NS = 16
EPT = E // NS
CHUNK = 80
NCH = EPT // CHUNK
NPAD = 10240
RPT = NPAD // NS
GRP = 25
NGRP = NCH // GRP
RB = 1000


def _node_tables(h, nt, wfr_ref, wbe_ref, war_ref, bar_ref, wself_ref,
                 bself_ref, base_ref, diff_ref, selfo_ref, alpha_ref):
    hfr = jnp.dot(h, wfr_ref[...], preferred_element_type=jnp.float32)
    hbe = jnp.dot(h, wbe_ref[...], preferred_element_type=jnp.float32)
    selfo_ref[...] = (
        jnp.dot(h, wself_ref[...], preferred_element_type=jnp.float32)
        + bself_ref[...])
    a_pre = jnp.sum(h * war_ref[...], axis=1, keepdims=True) + bar_ref[...][:, :1]
    a = jax.nn.sigmoid(a_pre)
    base_ref[...] = jnp.where(nt == 1, hfr, hbe)
    diff_ref[...] = jnp.where(nt == 2, hfr - hbe, jnp.zeros_like(hfr))
    alpha_ref[...] = jnp.broadcast_to(a, (h.shape[0], D))


def _dense_in_body(x_ref, nt_ref, wfr_ref, wbe_ref, war_ref, bar_ref,
                   wself_ref, bself_ref, base_ref, diff_ref, selfo_ref,
                   alpha_ref):
    _node_tables(x_ref[...], nt_ref[...], wfr_ref, wbe_ref, war_ref, bar_ref,
                 wself_ref, bself_ref, base_ref, diff_ref, selfo_ref, alpha_ref)


def _dense_mid_body(a_ref, b_ref, al_ref, so_ref, nt_ref, wfr_ref, wbe_ref,
                    war_ref, bar_ref, wself_ref, bself_ref, base_ref,
                    diff_ref, selfo_ref, alpha_ref):
    h = jnp.maximum(a_ref[...] + al_ref[...] * b_ref[...] + so_ref[...], 0.0)
    _node_tables(h, nt_ref[...], wfr_ref, wbe_ref, war_ref, bar_ref,
                 wself_ref, bself_ref, base_ref, diff_ref, selfo_ref, alpha_ref)


def _dense_out_body(a_ref, b_ref, al_ref, so_ref, wc_ref, bc_ref, out_ref):
    h = jnp.maximum(a_ref[...] + al_ref[...] * b_ref[...] + so_ref[...], 0.0)
    out_ref[...] = (
        jnp.dot(h, wc_ref[...], preferred_element_type=jnp.float32)
        + bc_ref[...])


_rowmap = lambda i: (i, 0)
_fixmap = lambda i: (0, 0)


def _dense_in(x, nt, wfr, wbe, war, bar, wself, bselfr):
    return pl.pallas_call(
        _dense_in_body,
        grid=(N // RB,),
        in_specs=[
            pl.BlockSpec((RB, D), _rowmap),
            pl.BlockSpec((RB, 1), _rowmap),
            pl.BlockSpec((D, D), _fixmap),
            pl.BlockSpec((D, D), _fixmap),
            pl.BlockSpec((1, D), _fixmap),
            pl.BlockSpec((1, D), _fixmap),
            pl.BlockSpec((D, D), _fixmap),
            pl.BlockSpec((1, D), _fixmap),
        ],
        out_specs=[pl.BlockSpec((RB, D), _rowmap)] * 4,
        out_shape=[jax.ShapeDtypeStruct((N, D), jnp.float32)] * 4,
    )(x, nt, wfr, wbe, war, bar, wself, bselfr)


def _dense_mid(a, b, al, so, nt, wfr, wbe, war, bar, wself, bselfr):
    return pl.pallas_call(
        _dense_mid_body,
        grid=(N // RB,),
        in_specs=[
            pl.BlockSpec((RB, D), _rowmap),
            pl.BlockSpec((RB, D), _rowmap),
            pl.BlockSpec((RB, D), _rowmap),
            pl.BlockSpec((RB, D), _rowmap),
            pl.BlockSpec((RB, 1), _rowmap),
            pl.BlockSpec((D, D), _fixmap),
            pl.BlockSpec((D, D), _fixmap),
            pl.BlockSpec((1, D), _fixmap),
            pl.BlockSpec((1, D), _fixmap),
            pl.BlockSpec((D, D), _fixmap),
            pl.BlockSpec((1, D), _fixmap),
        ],
        out_specs=[pl.BlockSpec((RB, D), _rowmap)] * 4,
        out_shape=[jax.ShapeDtypeStruct((N, D), jnp.float32)] * 4,
    )(a, b, al, so, nt, wfr, wbe, war, bar, wself, bselfr)


def _dense_out(a, b, al, so, wc, bcr):
    return pl.pallas_call(
        _dense_out_body,
        grid=(N // RB,),
        in_specs=[
            pl.BlockSpec((RB, D), _rowmap),
            pl.BlockSpec((RB, D), _rowmap),
            pl.BlockSpec((RB, D), _rowmap),
            pl.BlockSpec((RB, D), _rowmap),
            pl.BlockSpec((D, ODIM), _fixmap),
            pl.BlockSpec((1, ODIM), _fixmap),
        ],
        out_specs=pl.BlockSpec((RB, ODIM), _rowmap),
        out_shape=jax.ShapeDtypeStruct((N, ODIM), jnp.float32),
    )(a, b, al, so, wc, bcr)


def _sc_seg_body(base_hbm, diff_hbm, src_hbm, dst_hbm, zeros_hbm, a_hbm, b_hbm,
                 sidx, didx, rows, acc, sem):
    c = lax.axis_index("c")
    s = lax.axis_index("s")

    pltpu.sync_copy(zeros_hbm, acc.at[pl.ds(s * RPT, RPT)])
    plsc.subcore_barrier()

    def run(tab_hbm):
        @pl.loop(0, NGRP)
        def _(g):
            pltpu.sync_copy(src_hbm.at[s, g], sidx)
            pltpu.sync_copy(dst_hbm.at[s, g], didx)

            @pl.loop(0, GRP)
            def _(k):
                pltpu.async_copy(tab_hbm.at[sidx.at[k]], rows, sem).wait()
                pltpu.sync_copy(rows, acc.at[didx.at[k]], add=True)

    @pl.when(c == 0)
    def _():
        run(base_hbm)

    @pl.when(c == 1)
    def _():
        run(diff_hbm)

    plsc.subcore_barrier()

    @pl.when(c == 0)
    def _():
        pltpu.sync_copy(acc.at[pl.ds(s * RPT, RPT)],
                        a_hbm.at[pl.ds(s * RPT, RPT)])

    @pl.when(c == 1)
    def _():
        pltpu.sync_copy(acc.at[pl.ds(s * RPT, RPT)],
                        b_hbm.at[pl.ds(s * RPT, RPT)])


@functools.cache
def _sc_seg_build():
    mesh = plsc.VectorSubcoreMesh(core_axis_name="c", subcore_axis_name="s",
                                  num_cores=NC, num_subcores=NS)
    return pl.kernel(
        _sc_seg_body,
        out_type=(jax.ShapeDtypeStruct((NPAD, D), jnp.float32),
                  jax.ShapeDtypeStruct((NPAD, D), jnp.float32)),
        mesh=mesh,
        scratch_types=[
            pltpu.VMEM((GRP, CHUNK), jnp.int32),
            pltpu.VMEM((GRP, CHUNK), jnp.int32),
            pltpu.VMEM((CHUNK, D), jnp.float32),
            pltpu.VMEM_SHARED((NPAD, D), jnp.float32),
            pltpu.SemaphoreType.DMA,
        ],
    )


def _sc_seg(base, diff, src, dst, zeros):
    return _sc_seg_build()(base, diff, src, dst, zeros)


def kernel(x, edge_index, y, pmp_mask,
           W_fr1, W_be1, Wa1, ba1, W_self1, b_self1,
           W_fr2, W_be2, Wa2, ba2, W_self2, b_self2,
           Wc, bc):
    nt = jnp.where(pmp_mask, y.astype(jnp.int32), 2).astype(jnp.int32)[:, None]
    src = edge_index[0].astype(jnp.int32).reshape(NS, NGRP, GRP, CHUNK)
    dst = edge_index[1].astype(jnp.int32).reshape(NS, NGRP, GRP, CHUNK)
    zeros = jnp.zeros((RPT, D), jnp.float32)

    wa1r = Wa1.reshape(1, D)
    ba1r = jnp.broadcast_to(ba1.reshape(1, 1), (1, D))
    wa2r = Wa2.reshape(1, D)
    ba2r = jnp.broadcast_to(ba2.reshape(1, 1), (1, D))
    bs1r = b_self1.reshape(1, D)
    bs2r = b_self2.reshape(1, D)
    bcr = bc.reshape(1, ODIM)

    base1, diff1, self1, alpha1 = _dense_in(
        x, nt, W_fr1, W_be1, wa1r, ba1r, W_self1, bs1r)
    A1, B1 = _sc_seg(base1, diff1, src, dst, zeros)
    base2, diff2, self2, alpha2 = _dense_mid(
        A1, B1, alpha1, self1, nt, W_fr2, W_be2, wa2r, ba2r, W_self2, bs2r)
    A2, B2 = _sc_seg(base2, diff2, src, dst, zeros)
    return _dense_out(A2, B2, alpha2, self2, Wc, bcr)


# speedup vs baseline: 19.4418x; 1.5812x over previous
import functools

import jax
import jax.numpy as jnp
from jax import lax
from jax.experimental import pallas as pl
from jax.experimental.pallas import tpu as pltpu
from jax.experimental.pallas import tpu_sc as plsc

N = 10000
E = 320000
D = 128
ODIM = 16

NC = 2
NS = 16
EPT = E // NS
CHUNK = 80
NCH = EPT // CHUNK
NPAD = 10240
RPT = NPAD // NS
GRP = 25
NGRP = NCH // GRP
RB = 1000


def _node_tables(h, nt, wfr_ref, wbe_ref, war_ref, bar_ref, wself_ref,
                 bself_ref, base_ref, diff_ref, selfo_ref, alpha_ref):
    hfr = jnp.dot(h, wfr_ref[...], preferred_element_type=jnp.float32)
    hbe = jnp.dot(h, wbe_ref[...], preferred_element_type=jnp.float32)
    selfo_ref[...] = (
        jnp.dot(h, wself_ref[...], preferred_element_type=jnp.float32)
        + bself_ref[...])
    a_pre = jnp.sum(h * war_ref[...], axis=1, keepdims=True) + bar_ref[...][:, :1]
    a = jax.nn.sigmoid(a_pre)
    base_ref[...] = jnp.where(nt == 1, hfr, hbe)
    diff_ref[...] = jnp.where(nt == 2, hfr - hbe, jnp.zeros_like(hfr))
    alpha_ref[...] = jnp.broadcast_to(a, (h.shape[0], D))


def _dense_in_body(x_ref, nt_ref, wfr_ref, wbe_ref, war_ref, bar_ref,
                   wself_ref, bself_ref, base_ref, diff_ref, selfo_ref,
                   alpha_ref):
    _node_tables(x_ref[...], nt_ref[...], wfr_ref, wbe_ref, war_ref, bar_ref,
                 wself_ref, bself_ref, base_ref, diff_ref, selfo_ref, alpha_ref)


def _dense_mid_body(a_ref, b_ref, al_ref, so_ref, nt_ref, wfr_ref, wbe_ref,
                    war_ref, bar_ref, wself_ref, bself_ref, base_ref,
                    diff_ref, selfo_ref, alpha_ref):
    h = jnp.maximum(a_ref[...] + al_ref[...] * b_ref[...] + so_ref[...], 0.0)
    _node_tables(h, nt_ref[...], wfr_ref, wbe_ref, war_ref, bar_ref,
                 wself_ref, bself_ref, base_ref, diff_ref, selfo_ref, alpha_ref)


def _dense_out_body(a_ref, b_ref, al_ref, so_ref, wc_ref, bc_ref, out_ref):
    h = jnp.maximum(a_ref[...] + al_ref[...] * b_ref[...] + so_ref[...], 0.0)
    out_ref[...] = (
        jnp.dot(h, wc_ref[...], preferred_element_type=jnp.float32)
        + bc_ref[...])


_rowmap = lambda i: (i, 0)
_fixmap = lambda i: (0, 0)


def _dense_in(x, nt, wfr, wbe, war, bar, wself, bselfr):
    return pl.pallas_call(
        _dense_in_body,
        grid=(N // RB,),
        in_specs=[
            pl.BlockSpec((RB, D), _rowmap),
            pl.BlockSpec((RB, 1), _rowmap),
            pl.BlockSpec((D, D), _fixmap),
            pl.BlockSpec((D, D), _fixmap),
            pl.BlockSpec((1, D), _fixmap),
            pl.BlockSpec((1, D), _fixmap),
            pl.BlockSpec((D, D), _fixmap),
            pl.BlockSpec((1, D), _fixmap),
        ],
        out_specs=[pl.BlockSpec((RB, D), _rowmap)] * 4,
        out_shape=[jax.ShapeDtypeStruct((N, D), jnp.float32)] * 4,
    )(x, nt, wfr, wbe, war, bar, wself, bselfr)


def _dense_mid(a, b, al, so, nt, wfr, wbe, war, bar, wself, bselfr):
    return pl.pallas_call(
        _dense_mid_body,
        grid=(N // RB,),
        in_specs=[
            pl.BlockSpec((RB, D), _rowmap),
            pl.BlockSpec((RB, D), _rowmap),
            pl.BlockSpec((RB, D), _rowmap),
            pl.BlockSpec((RB, D), _rowmap),
            pl.BlockSpec((RB, 1), _rowmap),
            pl.BlockSpec((D, D), _fixmap),
            pl.BlockSpec((D, D), _fixmap),
            pl.BlockSpec((1, D), _fixmap),
            pl.BlockSpec((1, D), _fixmap),
            pl.BlockSpec((D, D), _fixmap),
            pl.BlockSpec((1, D), _fixmap),
        ],
        out_specs=[pl.BlockSpec((RB, D), _rowmap)] * 4,
        out_shape=[jax.ShapeDtypeStruct((N, D), jnp.float32)] * 4,
    )(a, b, al, so, nt, wfr, wbe, war, bar, wself, bselfr)


def _dense_out(a, b, al, so, wc, bcr):
    return pl.pallas_call(
        _dense_out_body,
        grid=(N // RB,),
        in_specs=[
            pl.BlockSpec((RB, D), _rowmap),
            pl.BlockSpec((RB, D), _rowmap),
            pl.BlockSpec((RB, D), _rowmap),
            pl.BlockSpec((RB, D), _rowmap),
            pl.BlockSpec((D, ODIM), _fixmap),
            pl.BlockSpec((1, ODIM), _fixmap),
        ],
        out_specs=pl.BlockSpec((RB, ODIM), _rowmap),
        out_shape=jax.ShapeDtypeStruct((N, ODIM), jnp.float32),
    )(a, b, al, so, wc, bcr)


def _sc_seg_body(base_hbm, diff_hbm, src_hbm, dst_hbm, zeros_hbm, a_hbm, b_hbm,
                 sidx, didx, rows0, rows1, acc, sem0, sem1):
    c = lax.axis_index("c")
    s = lax.axis_index("s")

    pltpu.sync_copy(zeros_hbm, acc.at[pl.ds(s * RPT, RPT)])
    plsc.subcore_barrier()

    def run(tab_hbm):
        def wait0():
            pltpu.make_async_copy(tab_hbm.at[sidx.at[0]], rows0, sem0).wait()

        def wait1():
            pltpu.make_async_copy(tab_hbm.at[sidx.at[0]], rows1, sem1).wait()

        @pl.loop(0, NGRP)
        def _(g):
            pltpu.sync_copy(src_hbm.at[s, g], sidx)
            pltpu.sync_copy(dst_hbm.at[s, g], didx)

            pltpu.async_copy(tab_hbm.at[sidx.at[0]], rows0, sem0)

            @pl.loop(0, GRP - 1, step=2)
            def _(k):
                pltpu.async_copy(tab_hbm.at[sidx.at[k + 1]], rows1, sem1)
                wait0()
                pltpu.sync_copy(rows0, acc.at[didx.at[k]], add=True)
                pltpu.async_copy(tab_hbm.at[sidx.at[k + 2]], rows0, sem0)
                wait1()
                pltpu.sync_copy(rows1, acc.at[didx.at[k + 1]], add=True)

            wait0()
            pltpu.sync_copy(rows0, acc.at[didx.at[GRP - 1]], add=True)

    @pl.when(c == 0)
    def _():
        run(base_hbm)

    @pl.when(c == 1)
    def _():
        run(diff_hbm)

    plsc.subcore_barrier()

    @pl.when(c == 0)
    def _():
        pltpu.sync_copy(acc.at[pl.ds(s * RPT, RPT)],
                        a_hbm.at[pl.ds(s * RPT, RPT)])

    @pl.when(c == 1)
    def _():
        pltpu.sync_copy(acc.at[pl.ds(s * RPT, RPT)],
                        b_hbm.at[pl.ds(s * RPT, RPT)])


@functools.cache
def _sc_seg_build():
    mesh = plsc.VectorSubcoreMesh(core_axis_name="c", subcore_axis_name="s",
                                  num_cores=NC, num_subcores=NS)
    return pl.kernel(
        _sc_seg_body,
        out_type=(jax.ShapeDtypeStruct((NPAD, D), jnp.float32),
                  jax.ShapeDtypeStruct((NPAD, D), jnp.float32)),
        mesh=mesh,
        scratch_types=[
            pltpu.VMEM((GRP, CHUNK), jnp.int32),
            pltpu.VMEM((GRP, CHUNK), jnp.int32),
            pltpu.VMEM((CHUNK, D), jnp.float32),
            pltpu.VMEM((CHUNK, D), jnp.float32),
            pltpu.VMEM_SHARED((NPAD, D), jnp.float32),
            pltpu.SemaphoreType.DMA,
            pltpu.SemaphoreType.DMA,
        ],
    )


def _sc_seg(base, diff, src, dst, zeros):
    return _sc_seg_build()(base, diff, src, dst, zeros)


def kernel(x, edge_index, y, pmp_mask,
           W_fr1, W_be1, Wa1, ba1, W_self1, b_self1,
           W_fr2, W_be2, Wa2, ba2, W_self2, b_self2,
           Wc, bc):
    nt = jnp.where(pmp_mask, y.astype(jnp.int32), 2).astype(jnp.int32)[:, None]
    src = edge_index[0].astype(jnp.int32).reshape(NS, NGRP, GRP, CHUNK)
    dst = edge_index[1].astype(jnp.int32).reshape(NS, NGRP, GRP, CHUNK)
    zeros = jnp.zeros((RPT, D), jnp.float32)

    wa1r = Wa1.reshape(1, D)
    ba1r = jnp.broadcast_to(ba1.reshape(1, 1), (1, D))
    wa2r = Wa2.reshape(1, D)
    ba2r = jnp.broadcast_to(ba2.reshape(1, 1), (1, D))
    bs1r = b_self1.reshape(1, D)
    bs2r = b_self2.reshape(1, D)
    bcr = bc.reshape(1, ODIM)

    base1, diff1, self1, alpha1 = _dense_in(
        x, nt, W_fr1, W_be1, wa1r, ba1r, W_self1, bs1r)
    A1, B1 = _sc_seg(base1, diff1, src, dst, zeros)
    base2, diff2, self2, alpha2 = _dense_mid(
        A1, B1, alpha1, self1, nt, W_fr2, W_be2, wa2r, ba2r, W_self2, bs2r)
    A2, B2 = _sc_seg(base2, diff2, src, dst, zeros)
    return _dense_out(A2, B2, alpha2, self2, Wc, bcr)
